# baseline (speedup 1.0000x reference)
import functools

import jax
import jax.numpy as jnp
from jax import lax
from jax.experimental import pallas as pl
from jax.experimental.pallas import tpu as pltpu
from jax.experimental.pallas import tpu_sc as plsc

_V = 8192
_D = 8192
_NB = 8192
_NC = 2
_NS = 16
_NW = _NC * _NS
_BPW = _NB // _NW
_L = 16
_NG = _BPW // _L


@functools.partial(
    pl.kernel,
    out_type=jax.ShapeDtypeStruct((_NB, _D), jnp.float32),
    mesh=plsc.VectorSubcoreMesh(core_axis_name="c", subcore_axis_name="s"),
    scratch_types=[
        pltpu.VMEM((_BPW,), jnp.int32),
        pltpu.SemaphoreType.DMA,
    ],
)
def _gather_rows(x_hbm, table_hbm, out_hbm, idx_v, sem):
    wid = lax.axis_index("s") * _NC + lax.axis_index("c")
    base = wid * _BPW
    pltpu.sync_copy(x_hbm.at[wid], idx_v)

    def fire(g, carry):
        vec = idx_v[pl.ds(g * _L, _L)]
        for lane in range(_L):
            pltpu.async_copy(
                table_hbm.at[pl.ds(vec[lane], 1)],
                out_hbm.at[pl.ds(base + g * _L + lane, 1)],
                sem,
            )
        return carry

    def drain(g, carry):
        for lane in range(_L):
            pltpu.make_async_copy(
                table_hbm.at[pl.ds(0, 1)],
                out_hbm.at[pl.ds(base, 1)],
                sem,
            ).wait()
        return carry

    fire(0, 0)

    def body(g, carry):
        fire(g + 1, carry)
        drain(g, carry)
        return carry

    lax.fori_loop(0, _NG - 1, body, 0)
    drain(_NG - 1, 0)


def kernel(x, table):
    x2 = x.reshape(_NW, _BPW).astype(jnp.int32)
    out = _gather_rows(x2, table)
    return out.reshape(x.shape[0], x.shape[1], _D)


# speedup vs baseline: 39.5072x; 39.5072x over previous
import functools

import jax
import jax.numpy as jnp
from jax import lax
from jax.experimental import pallas as pl
from jax.experimental.pallas import tpu as pltpu
from jax.experimental.pallas import tpu_sc as plsc

_V = 8192
_D = 8192
_NB = 8192
_NC = 2
_NS = 16
_NW = _NC * _NS
_BPW = _NB // _NW
_K = 4
_NCHUNK = _BPW // _K
_NBUF = 3


@functools.partial(
    pl.kernel,
    out_type=jax.ShapeDtypeStruct((_NB, _D), jnp.float32),
    mesh=plsc.VectorSubcoreMesh(core_axis_name="c", subcore_axis_name="s"),
    scratch_types=[
        pltpu.VMEM((_NCHUNK, _K), jnp.int32),
        pltpu.VMEM((_K, _D), jnp.float32),
        pltpu.VMEM((_K, _D), jnp.float32),
        pltpu.VMEM((_K, _D), jnp.float32),
        pltpu.SemaphoreType.DMA,
        pltpu.SemaphoreType.DMA,
        pltpu.SemaphoreType.DMA,
        pltpu.SemaphoreType.DMA,
        pltpu.SemaphoreType.DMA,
        pltpu.SemaphoreType.DMA,
    ],
)
def _gather_rows(
    x_hbm, table_hbm, out_hbm, idx_v, buf0, buf1, buf2, g0, g1, g2, s0, s1, s2
):
    wid = lax.axis_index("s") * _NC + lax.axis_index("c")
    base = wid * _BPW
    pltpu.sync_copy(x_hbm.at[wid], idx_v)

    bufs = (buf0, buf1, buf2)
    gsems = (g0, g1, g2)
    ssems = (s0, s1, s2)

    def gather_start(cur, b):
        pltpu.async_copy(table_hbm.at[idx_v.at[cur]], bufs[b], gsems[b])

    def gather_wait(cur, b):
        pltpu.make_async_copy(table_hbm.at[idx_v.at[cur]], bufs[b], gsems[b]).wait()

    def scatter_start(cur, b):
        pltpu.async_copy(bufs[b], out_hbm.at[pl.ds(base + cur * _K, _K)], ssems[b])

    def scatter_wait(cur, b):
        pltpu.make_async_copy(
            bufs[b], out_hbm.at[pl.ds(base + cur * _K, _K)], ssems[b]
        ).wait()

    gather_start(0, 0)
    gather_start(1, 1)

    def body(i, carry):
        for b in range(_NBUF):
            c = i * _NBUF + b

            cb = b
            nb = (b + 2) % _NBUF

            @pl.when(c < _NCHUNK)
            def _():
                gather_wait(c, cb)
                scatter_start(c, cb)

                @pl.when(c >= 1)
                def _():
                    scatter_wait(c - 1, nb)

                @pl.when(c + 2 < _NCHUNK)
                def _():
                    gather_start(c + 2, nb)

        return carry

    lax.fori_loop(0, (_NCHUNK + _NBUF - 1) // _NBUF, body, 0)
    scatter_wait(_NCHUNK - 1, (_NCHUNK - 1) % _NBUF)


def kernel(x, table):
    x3 = x.reshape(_NW, _NCHUNK, _K).astype(jnp.int32)
    out = _gather_rows(x3, table)
    return out.reshape(x.shape[0], x.shape[1], _D)
